# baseline (speedup 1.0000x reference)
import functools

import jax
import jax.numpy as jnp
from jax import lax
from jax.experimental import pallas as pl
from jax.experimental.pallas import tpu as pltpu
from jax.experimental.pallas import tpu_sc as plsc

N = 10000
E = 320000
D_IN = 128
D_HID = 8
N_CLASSES = 40

NC = 2
NS = 16
NW = NC * NS

CHUNK = 512
CPW = 20
E_PAD = NW * CPW * CHUNK
TW = 16
ACC_ROWS = 10112
DUMMY = N
ZROWS = ACC_ROWS // NS

Q = 2
ROWS_BLK = ACC_ROWS // 16
GRID = 16
OUT_BLK = 1000
OUT_GRID = N // OUT_BLK


@functools.cache
def _segment_accumulate_sc():
  mesh = plsc.VectorSubcoreMesh(
      core_axis_name="c", subcore_axis_name="s", num_cores=NC,
      num_subcores=NS)

  @functools.partial(
      pl.kernel,
      out_type=jax.ShapeDtypeStruct((NC, ACC_ROWS, TW), jnp.float32),
      mesh=mesh,
      scratch_types=[
          pltpu.VMEM((CPW, CHUNK), jnp.int32),
          pltpu.VMEM((CPW, CHUNK), jnp.int32),
          [pltpu.VMEM((CHUNK, TW), jnp.float32) for _ in range(2 * Q)],
          pltpu.VMEM_SHARED((ACC_ROWS, TW), jnp.float32),
          pltpu.VMEM_SHARED((ACC_ROWS, TW), jnp.float32),
          [pltpu.SemaphoreType.DMA for _ in range(4)],
      ],
      compiler_params=pltpu.CompilerParams(use_tc_tiling_on_sc=False),
  )
  def seg_acc(table_hbm, src_hbm, dst_hbm, zeros_hbm, out_hbm,
              sidx_v, didx_v, bufs, tbl_sh, acc_sh, sems):
    c = lax.axis_index("c")
    s = lax.axis_index("s")
    wid = s * NC + c
    grp = (tuple(bufs[:Q]), tuple(bufs[Q:]))
    gsem = (sems[0], sems[1])
    ssem = (sems[2], sems[3])
    pltpu.sync_copy(table_hbm.at[pl.ds(s * ZROWS, ZROWS)],
                    tbl_sh.at[pl.ds(s * ZROWS, ZROWS)])
    pltpu.sync_copy(zeros_hbm.at[pl.ds(s * ZROWS, ZROWS)],
                    acc_sh.at[pl.ds(s * ZROWS, ZROWS)])
    pltpu.sync_copy(src_hbm.at[wid], sidx_v)
    pltpu.sync_copy(dst_hbm.at[wid], didx_v)
    plsc.subcore_barrier()

    def fire_gathers(g, jbase):
      for i in range(Q):
        jn = jnp.minimum(jbase + i, CPW - 1)
        pltpu.async_copy(tbl_sh.at[sidx_v.at[jn]], grp[g][i], gsem[g])

    def drain_gathers(g):
      for i in range(Q):
        pltpu.make_async_copy(tbl_sh.at[sidx_v.at[0]], grp[g][i],
                              gsem[g]).wait()

    def fire_scatters(g, jbase):
      for i in range(Q):
        pltpu.async_copy(grp[g][i], acc_sh.at[didx_v.at[jbase + i]],
                         ssem[g], add=True)

    def drain_scatters(g):
      for i in range(Q):
        pltpu.make_async_copy(grp[g][i], acc_sh.at[didx_v.at[0]],
                              ssem[g]).wait()

    fire_gathers(0, 0)

    def round_fn(r, carry):
      j0 = r * 2 * Q

      @pl.when(r > 0)
      def _():
        drain_scatters(1)
      fire_gathers(1, j0 + Q)
      drain_gathers(0)
      fire_scatters(0, j0)

      drain_scatters(0)
      fire_gathers(0, j0 + 2 * Q)
      drain_gathers(1)
      fire_scatters(1, j0 + Q)
      return carry

    lax.fori_loop(0, CPW // (2 * Q), round_fn, 0)
    drain_scatters(1)
    drain_gathers(0)
    plsc.subcore_barrier()
    pltpu.sync_copy(acc_sh.at[pl.ds(s * ZROWS, ZROWS)],
                    out_hbm.at[c, pl.ds(s * ZROWS, ZROWS)])

  return seg_acc


def _l1_body(x_ref, w_ref, b_ref, t_ref, q_ref):
  h = jnp.dot(x_ref[...], w_ref[...],
              preferred_element_type=jnp.float32,
              precision=lax.Precision.HIGHEST)
  ones = jnp.ones((ROWS_BLK, 1), jnp.float32)
  zeros = jnp.zeros((ROWS_BLK, TW - D_HID - 1), jnp.float32)
  t_ref[...] = jnp.concatenate([h[:, 0:D_HID], ones, zeros], axis=1)
  q_ref[...] = h[:, D_HID:2 * D_HID] + b_ref[...]


def _l2_body(parts_ref, q_ref, w2r_ref, b2_ref, t2_ref, hr2_ref):
  ssum = parts_ref[0] + parts_ref[1]
  deg = ssum[:, D_HID:D_HID + 1]
  rdeg = 1.0 / jnp.maximum(deg, 1.0)
  h1 = jnp.maximum(ssum[:, 0:D_HID] * rdeg + q_ref[...], 0.0)
  zeros = jnp.zeros((ROWS_BLK, TW - D_HID - 1), jnp.float32)
  t2_ref[...] = jnp.concatenate([h1, rdeg, zeros], axis=1)
  hr2_ref[...] = jnp.dot(h1, w2r_ref[...],
                         preferred_element_type=jnp.float32,
                         precision=lax.Precision.HIGHEST) + b2_ref[...]


def _out_body(parts_ref, t2_ref, hr2_ref, w2l_ref, o_ref):
  ssum = parts_ref[0] + parts_ref[1]
  rdeg = t2_ref[:, D_HID:D_HID + 1]
  agg2 = ssum[:, 0:D_HID] * rdeg
  logits = jnp.dot(agg2, w2l_ref[...],
                   preferred_element_type=jnp.float32,
                   precision=lax.Precision.HIGHEST) + hr2_ref[...]
  m = jnp.max(logits, axis=1, keepdims=True)
  z = logits - m
  lse = jnp.log(jnp.sum(jnp.exp(z), axis=1, keepdims=True))
  o_ref[...] = z - lse


_row_spec = lambda blk, w: pl.BlockSpec((blk, w), lambda i: (i, 0))
_full_spec = lambda r, w: pl.BlockSpec((r, w), lambda i: (0, 0))
_parts_spec = lambda blk: pl.BlockSpec((NC, blk, TW), lambda i: (0, i, 0))

_l1_call = pl.pallas_call(
    _l1_body,
    grid=(GRID,),
    in_specs=[_row_spec(ROWS_BLK, D_IN), _full_spec(D_IN, TW),
              _full_spec(1, D_HID)],
    out_specs=[_row_spec(ROWS_BLK, TW), _row_spec(ROWS_BLK, D_HID)],
    out_shape=[jax.ShapeDtypeStruct((ACC_ROWS, TW), jnp.float32),
               jax.ShapeDtypeStruct((ACC_ROWS, D_HID), jnp.float32)],
)

_l2_call = pl.pallas_call(
    _l2_body,
    grid=(GRID,),
    in_specs=[_parts_spec(ROWS_BLK), _row_spec(ROWS_BLK, D_HID),
              _full_spec(D_HID, N_CLASSES), _full_spec(1, N_CLASSES)],
    out_specs=[_row_spec(ROWS_BLK, TW), _row_spec(ROWS_BLK, N_CLASSES)],
    out_shape=[jax.ShapeDtypeStruct((ACC_ROWS, TW), jnp.float32),
               jax.ShapeDtypeStruct((ACC_ROWS, N_CLASSES), jnp.float32)],
)

_out_call = pl.pallas_call(
    _out_body,
    grid=(OUT_GRID,),
    in_specs=[_parts_spec(OUT_BLK), _row_spec(OUT_BLK, TW),
              _row_spec(OUT_BLK, N_CLASSES), _full_spec(D_HID, N_CLASSES)],
    out_specs=_row_spec(OUT_BLK, N_CLASSES),
    out_shape=jax.ShapeDtypeStruct((N, N_CLASSES), jnp.float32),
)


def kernel(x, edge_index, W1l, W1r, b1, W2l, W2r, b2):
  src = edge_index[0].astype(jnp.int32)
  dst = edge_index[1].astype(jnp.int32)
  pad = E_PAD - E
  src_p = jnp.concatenate([src, jnp.zeros((pad,), jnp.int32)])
  dst_p = jnp.concatenate([dst, jnp.full((pad,), DUMMY, jnp.int32)])
  src_p = src_p.reshape(NW, CPW, CHUNK)
  dst_p = dst_p.reshape(NW, CPW, CHUNK)
  zeros_acc = jnp.zeros((ACC_ROWS, TW), jnp.float32)

  W1 = jnp.concatenate([W1l, W1r], axis=1)

  seg_acc = _segment_accumulate_sc()
  t1, q1b = _l1_call(x, W1, b1.reshape(1, D_HID))
  parts1 = seg_acc(t1, src_p, dst_p, zeros_acc)
  t2, hr2 = _l2_call(parts1, q1b, W2r, b2.reshape(1, N_CLASSES))
  parts2 = seg_acc(t2, src_p, dst_p, zeros_acc)
  return _out_call(parts2, t2, hr2, W2l)


# speedup vs baseline: 1.0827x; 1.0827x over previous
import functools

import jax
import jax.numpy as jnp
from jax import lax
from jax.experimental import pallas as pl
from jax.experimental.pallas import tpu as pltpu
from jax.experimental.pallas import tpu_sc as plsc

N = 10000
E = 320000
D_IN = 128
D_HID = 8
N_CLASSES = 40

NC = 2
NS = 16
NW = NC * NS
L = 16

EPW = E // NW
CHUNK = 512
CPW = 20
FULLW = (EPW // CHUNK) * CHUNK
TAILW = EPW - FULLW
SLOTS = CPW * CHUNK
TW = 16
ACC_ROWS = 10112
DUMMY = N
ZROWS = ACC_ROWS // NS

Q = 2
ROWS_BLK = ACC_ROWS // 16
GRID = 16
OUT_BLK = 1000
OUT_GRID = N // OUT_BLK

_MESH = dict(core_axis_name="c", subcore_axis_name="s", num_cores=NC,
             num_subcores=NS)
_SC_PARAMS = pltpu.CompilerParams(use_tc_tiling_on_sc=False,
                                  needs_layout_passes=False)


def _stage_indices(edge_hbm, sidx_v, didx_v, wid):
  base = wid * EPW
  pltpu.sync_copy(edge_hbm.at[0, pl.ds(base, FULLW)],
                  sidx_v.at[pl.ds(0, FULLW)])
  pltpu.sync_copy(edge_hbm.at[0, pl.ds(base + FULLW, TAILW)],
                  sidx_v.at[pl.ds(FULLW, TAILW)])
  pltpu.sync_copy(edge_hbm.at[1, pl.ds(base, FULLW)],
                  didx_v.at[pl.ds(0, FULLW)])
  pltpu.sync_copy(edge_hbm.at[1, pl.ds(base + FULLW, TAILW)],
                  didx_v.at[pl.ds(FULLW, TAILW)])
  for k in range(EPW, SLOTS, L):
    sidx_v[pl.ds(k, L)] = jnp.zeros((L,), jnp.int32)
    didx_v[pl.ds(k, L)] = jnp.full((L,), DUMMY, jnp.int32)


def _edge_loop(sidx_v, didx_v, bufs, sems, tbl_sh, acc_sh):
  grp = (tuple(bufs[:Q]), tuple(bufs[Q:]))
  gsem = (sems[0], sems[1])
  ssem = (sems[2], sems[3])

  def islice(iv, j):
    return iv.at[pl.ds(pl.multiple_of(j * CHUNK, CHUNK), CHUNK)]

  def fire_gathers(g, jbase):
    for i in range(Q):
      jn = jnp.minimum(jbase + i, CPW - 1)
      pltpu.async_copy(tbl_sh.at[islice(sidx_v, jn)], grp[g][i], gsem[g])

  def drain_gathers(g):
    for i in range(Q):
      pltpu.make_async_copy(tbl_sh.at[islice(sidx_v, 0)], grp[g][i],
                            gsem[g]).wait()

  def fire_scatters(g, jbase):
    for i in range(Q):
      pltpu.async_copy(grp[g][i], acc_sh.at[islice(didx_v, jbase + i)],
                       ssem[g], add=True)

  def drain_scatters(g):
    for i in range(Q):
      pltpu.make_async_copy(grp[g][i], acc_sh.at[islice(didx_v, 0)],
                            ssem[g]).wait()

  fire_gathers(0, 0)

  def round_fn(r, carry):
    j0 = r * 2 * Q

    @pl.when(r > 0)
    def _():
      drain_scatters(1)
    fire_gathers(1, j0 + Q)
    drain_gathers(0)
    fire_scatters(0, j0)

    drain_scatters(0)
    fire_gathers(0, j0 + 2 * Q)
    drain_gathers(1)
    fire_scatters(1, j0 + Q)
    return carry

  lax.fori_loop(0, CPW // (2 * Q), round_fn, 0)
  drain_scatters(1)
  drain_gathers(0)


_IDX_SCRATCH = [pltpu.VMEM((SLOTS,), jnp.int32),
                pltpu.VMEM((SLOTS,), jnp.int32)]
_BUF_SCRATCH = [pltpu.VMEM((CHUNK, TW), jnp.float32) for _ in range(2 * Q)]
_SEM_SCRATCH = [pltpu.SemaphoreType.DMA for _ in range(4)]


@functools.cache
def _sc_layer1():
  @functools.partial(
      pl.kernel,
      out_type=jax.ShapeDtypeStruct((NC, ACC_ROWS, TW), jnp.float32),
      mesh=plsc.VectorSubcoreMesh(**_MESH),
      scratch_types=[
          *_IDX_SCRATCH, _BUF_SCRATCH,
          pltpu.VMEM_SHARED((ACC_ROWS, TW), jnp.float32),
          pltpu.VMEM_SHARED((ACC_ROWS, TW), jnp.float32),
          _SEM_SCRATCH,
      ],
      compiler_params=_SC_PARAMS,
  )
  def sc1(table_hbm, edge_hbm, zeros_hbm, out_hbm,
          sidx_v, didx_v, bufs, tbl_sh, acc_sh, sems):
    c = lax.axis_index("c")
    s = lax.axis_index("s")
    wid = s * NC + c
    stripe = pl.ds(s * ZROWS, ZROWS)
    pltpu.sync_copy(table_hbm.at[stripe], tbl_sh.at[stripe])
    pltpu.sync_copy(zeros_hbm.at[stripe], acc_sh.at[stripe])
    _stage_indices(edge_hbm, sidx_v, didx_v, wid)
    plsc.subcore_barrier()
    _edge_loop(sidx_v, didx_v, bufs, sems, tbl_sh, acc_sh)
    plsc.subcore_barrier()
    pltpu.sync_copy(acc_sh.at[stripe], out_hbm.at[c, stripe])

  return sc1


@functools.cache
def _sc_layer2():
  @functools.partial(
      pl.kernel,
      out_type=(jax.ShapeDtypeStruct((NC, ACC_ROWS, TW), jnp.float32),
                jax.ShapeDtypeStruct((ACC_ROWS, TW), jnp.float32)),
      mesh=plsc.VectorSubcoreMesh(**_MESH),
      scratch_types=[
          *_IDX_SCRATCH, _BUF_SCRATCH,
          pltpu.VMEM_SHARED((ACC_ROWS, TW), jnp.float32),
          pltpu.VMEM_SHARED((ACC_ROWS, TW), jnp.float32),
          _SEM_SCRATCH,
          pltpu.VMEM((ZROWS, TW), jnp.float32),
          pltpu.VMEM((ZROWS, TW), jnp.float32),
          pltpu.VMEM((ZROWS, TW), jnp.float32),
          pltpu.VMEM((ZROWS, TW), jnp.float32),
      ],
      compiler_params=_SC_PARAMS,
  )
  def sc2(parts1_hbm, q1_hbm, edge_hbm, zeros_hbm, out_hbm, t2_hbm,
          sidx_v, didx_v, bufs, tbl_sh, acc_sh, sems,
          ma_v, mb_v, mq_v, mt_v):
    c = lax.axis_index("c")
    s = lax.axis_index("s")
    wid = s * NC + c
    stripe = pl.ds(s * ZROWS, ZROWS)
    pltpu.sync_copy(zeros_hbm.at[stripe], acc_sh.at[stripe])
    _stage_indices(edge_hbm, sidx_v, didx_v, wid)
    pltpu.sync_copy(parts1_hbm.at[0, stripe], ma_v)
    pltpu.sync_copy(parts1_hbm.at[1, stripe], mb_v)
    pltpu.sync_copy(q1_hbm.at[stripe], mq_v)
    pltpu.sync_copy(zeros_hbm.at[stripe], mt_v)

    iota = lax.broadcasted_iota(jnp.int32, (L,), 0)

    def block_fn(blk, carry):
      ridx = jnp.minimum(blk * L + iota, ZROWS - 1)

      def col(ref, j):
        return plsc.load_gather(ref, [ridx, jnp.full((L,), j, jnp.int32)])

      deg = col(ma_v, D_HID) + col(mb_v, D_HID)
      rdeg = 1.0 / jnp.maximum(deg, 1.0)
      plsc.store_scatter(mt_v, [ridx, jnp.full((L,), D_HID, jnp.int32)],
                         rdeg)
      for j in range(D_HID):
        agg = (col(ma_v, j) + col(mb_v, j)) * rdeg
        h1 = jnp.maximum(agg + col(mq_v, j), 0.0)
        plsc.store_scatter(mt_v, [ridx, jnp.full((L,), j, jnp.int32)], h1)
      return carry

    lax.fori_loop(0, (ZROWS + L - 1) // L, block_fn, 0)
    pltpu.sync_copy(mt_v, tbl_sh.at[stripe])

    @pl.when(c == 0)
    def _():
      pltpu.sync_copy(mt_v, t2_hbm.at[stripe])

    plsc.subcore_barrier()
    _edge_loop(sidx_v, didx_v, bufs, sems, tbl_sh, acc_sh)
    plsc.subcore_barrier()
    pltpu.sync_copy(acc_sh.at[stripe], out_hbm.at[c, stripe])

  return sc2


def _l1_body(x_ref, wl_ref, wr_ref, b_ref, t_ref, q_ref):
  w = jnp.concatenate([wl_ref[...], wr_ref[...]], axis=1)
  h = jnp.dot(x_ref[...], w, preferred_element_type=jnp.float32,
              precision=lax.Precision.HIGHEST)
  ones = jnp.ones((ROWS_BLK, 1), jnp.float32)
  zeros7 = jnp.zeros((ROWS_BLK, TW - D_HID - 1), jnp.float32)
  t_ref[...] = jnp.concatenate([h[:, 0:D_HID], ones, zeros7], axis=1)
  zeros8 = jnp.zeros((ROWS_BLK, TW - D_HID), jnp.float32)
  q_ref[...] = jnp.concatenate(
      [h[:, D_HID:2 * D_HID] + b_ref[...], zeros8], axis=1)


def _out_body(parts_ref, t2_ref, w2l_ref, w2r_ref, b2_ref, o_ref):
  ssum = parts_ref[0] + parts_ref[1]
  rdeg = t2_ref[:, D_HID:D_HID + 1]
  h1 = t2_ref[:, 0:D_HID]
  agg2 = ssum[:, 0:D_HID] * rdeg
  logits = (jnp.dot(agg2, w2l_ref[...], preferred_element_type=jnp.float32,
                    precision=lax.Precision.HIGHEST)
            + jnp.dot(h1, w2r_ref[...], preferred_element_type=jnp.float32,
                      precision=lax.Precision.HIGHEST)
            + b2_ref[...])
  m = jnp.max(logits, axis=1, keepdims=True)
  z = logits - m
  lse = jnp.log(jnp.sum(jnp.exp(z), axis=1, keepdims=True))
  o_ref[...] = z - lse


_row_spec = lambda blk, w: pl.BlockSpec((blk, w), lambda i: (i, 0))
_full_spec = lambda r, w: pl.BlockSpec((r, w), lambda i: (0, 0))
_parts_spec = lambda blk: pl.BlockSpec((NC, blk, TW), lambda i: (0, i, 0))

_l1_call = pl.pallas_call(
    _l1_body,
    grid=(GRID,),
    in_specs=[_row_spec(ROWS_BLK, D_IN), _full_spec(D_IN, D_HID),
              _full_spec(D_IN, D_HID), _full_spec(1, D_HID)],
    out_specs=[_row_spec(ROWS_BLK, TW), _row_spec(ROWS_BLK, TW)],
    out_shape=[jax.ShapeDtypeStruct((ACC_ROWS, TW), jnp.float32),
               jax.ShapeDtypeStruct((ACC_ROWS, TW), jnp.float32)],
)

_out_call = pl.pallas_call(
    _out_body,
    grid=(OUT_GRID,),
    in_specs=[_parts_spec(OUT_BLK), _row_spec(OUT_BLK, TW),
              _full_spec(D_HID, N_CLASSES), _full_spec(D_HID, N_CLASSES),
              _full_spec(1, N_CLASSES)],
    out_specs=_row_spec(OUT_BLK, N_CLASSES),
    out_shape=jax.ShapeDtypeStruct((N, N_CLASSES), jnp.float32),
)


def kernel(x, edge_index, W1l, W1r, b1, W2l, W2r, b2):
  edges = edge_index.astype(jnp.int32)
  zeros_acc = jnp.zeros((ACC_ROWS, TW), jnp.float32)
  t1, q1 = _l1_call(x, W1l, W1r, b1.reshape(1, D_HID))
  parts1 = _sc_layer1()(t1, edges, zeros_acc)
  parts2, t2 = _sc_layer2()(parts1, q1, edges, zeros_acc)
  return _out_call(parts2, t2, W2l, W2r, b2.reshape(1, N_CLASSES))


# speedup vs baseline: 1.1471x; 1.0595x over previous
import functools

import jax
import jax.numpy as jnp
from jax import lax
from jax.experimental import pallas as pl
from jax.experimental.pallas import tpu as pltpu
from jax.experimental.pallas import tpu_sc as plsc

N = 10000
E = 320000
D_IN = 128
D_HID = 8
N_CLASSES = 40

NC = 2
NS = 16
NW = NC * NS
L = 16

EPW = E // NW
CHUNK = 512
CPW = 20
FULLW = (EPW // CHUNK) * CHUNK
TAILW = EPW - FULLW
SLOTS = CPW * CHUNK
TW = 16
ACC_ROWS = 10112
DUMMY = N
ZROWS = ACC_ROWS // NS

Q = 2
GRID = 4
ROWS_BLK = ACC_ROWS // GRID
OUT_GRID = 2
OUT_BLK = N // OUT_GRID

_MESH = dict(core_axis_name="c", subcore_axis_name="s", num_cores=NC,
             num_subcores=NS)
_SC_PARAMS = pltpu.CompilerParams(use_tc_tiling_on_sc=False,
                                  needs_layout_passes=False)


def _stage_indices(edge_hbm, sidx_v, didx_v, wid):
  base = wid * EPW
  pltpu.sync_copy(edge_hbm.at[0, pl.ds(base, FULLW)],
                  sidx_v.at[pl.ds(0, FULLW)])
  pltpu.sync_copy(edge_hbm.at[0, pl.ds(base + FULLW, TAILW)],
                  sidx_v.at[pl.ds(FULLW, TAILW)])
  pltpu.sync_copy(edge_hbm.at[1, pl.ds(base, FULLW)],
                  didx_v.at[pl.ds(0, FULLW)])
  pltpu.sync_copy(edge_hbm.at[1, pl.ds(base + FULLW, TAILW)],
                  didx_v.at[pl.ds(FULLW, TAILW)])
  for k in range(EPW, SLOTS, L):
    sidx_v[pl.ds(k, L)] = jnp.zeros((L,), jnp.int32)
    didx_v[pl.ds(k, L)] = jnp.full((L,), DUMMY, jnp.int32)


def _edge_loop(sidx_v, didx_v, bufs, sems, tbl_sh, acc_sh):
  grp = (tuple(bufs[:Q]), tuple(bufs[Q:]))
  gsem = (sems[0], sems[1])
  ssem = (sems[2], sems[3])

  def islice(iv, j):
    return iv.at[pl.ds(pl.multiple_of(j * CHUNK, CHUNK), CHUNK)]

  def fire_gathers(g, jbase):
    for i in range(Q):
      jn = jnp.minimum(jbase + i, CPW - 1)
      pltpu.async_copy(tbl_sh.at[islice(sidx_v, jn)], grp[g][i], gsem[g])

  def drain_gathers(g):
    for i in range(Q):
      pltpu.make_async_copy(tbl_sh.at[islice(sidx_v, 0)], grp[g][i],
                            gsem[g]).wait()

  def fire_scatters(g, jbase):
    for i in range(Q):
      pltpu.async_copy(grp[g][i], acc_sh.at[islice(didx_v, jbase + i)],
                       ssem[g], add=True)

  def drain_scatters(g):
    for i in range(Q):
      pltpu.make_async_copy(grp[g][i], acc_sh.at[islice(didx_v, 0)],
                            ssem[g]).wait()

  fire_gathers(0, 0)

  def round_fn(r, carry):
    j0 = r * 2 * Q

    @pl.when(r > 0)
    def _():
      drain_scatters(1)
    fire_gathers(1, j0 + Q)
    drain_gathers(0)
    fire_scatters(0, j0)

    drain_scatters(0)
    fire_gathers(0, j0 + 2 * Q)
    drain_gathers(1)
    fire_scatters(1, j0 + Q)
    return carry

  lax.fori_loop(0, CPW // (2 * Q), round_fn, 0)
  drain_scatters(1)
  drain_gathers(0)


_IDX_SCRATCH = [pltpu.VMEM((SLOTS,), jnp.int32),
                pltpu.VMEM((SLOTS,), jnp.int32)]
_BUF_SCRATCH = [pltpu.VMEM((CHUNK, TW), jnp.float32) for _ in range(2 * Q)]
_SEM_SCRATCH = [pltpu.SemaphoreType.DMA for _ in range(4)]


@functools.cache
def _sc_layer1():
  @functools.partial(
      pl.kernel,
      out_type=jax.ShapeDtypeStruct((NC, ACC_ROWS, TW), jnp.float32),
      mesh=plsc.VectorSubcoreMesh(**_MESH),
      scratch_types=[
          *_IDX_SCRATCH, _BUF_SCRATCH,
          pltpu.VMEM_SHARED((ACC_ROWS, TW), jnp.float32),
          pltpu.VMEM_SHARED((ACC_ROWS, TW), jnp.float32),
          _SEM_SCRATCH,
      ],
      compiler_params=_SC_PARAMS,
  )
  def sc1(table_hbm, edge_hbm, zeros_hbm, out_hbm,
          sidx_v, didx_v, bufs, tbl_sh, acc_sh, sems):
    c = lax.axis_index("c")
    s = lax.axis_index("s")
    wid = s * NC + c
    stripe = pl.ds(s * ZROWS, ZROWS)
    pltpu.sync_copy(table_hbm.at[stripe], tbl_sh.at[stripe])
    pltpu.sync_copy(zeros_hbm.at[stripe], acc_sh.at[stripe])
    _stage_indices(edge_hbm, sidx_v, didx_v, wid)
    plsc.subcore_barrier()
    _edge_loop(sidx_v, didx_v, bufs, sems, tbl_sh, acc_sh)
    plsc.subcore_barrier()
    pltpu.sync_copy(acc_sh.at[stripe], out_hbm.at[c, stripe])

  return sc1


@functools.cache
def _sc_layer2():
  @functools.partial(
      pl.kernel,
      out_type=(jax.ShapeDtypeStruct((NC, ACC_ROWS, TW), jnp.float32),
                jax.ShapeDtypeStruct((ACC_ROWS, TW), jnp.float32)),
      mesh=plsc.VectorSubcoreMesh(**_MESH),
      scratch_types=[
          *_IDX_SCRATCH, _BUF_SCRATCH,
          pltpu.VMEM_SHARED((ACC_ROWS, TW), jnp.float32),
          pltpu.VMEM_SHARED((ACC_ROWS, TW), jnp.float32),
          _SEM_SCRATCH,
          pltpu.VMEM((ZROWS, TW), jnp.float32),
          pltpu.VMEM((ZROWS, TW), jnp.float32),
          pltpu.VMEM((ZROWS, TW), jnp.float32),
          pltpu.VMEM((ZROWS, TW), jnp.float32),
      ],
      compiler_params=_SC_PARAMS,
  )
  def sc2(parts1_hbm, q1_hbm, edge_hbm, zeros_hbm, out_hbm, t2_hbm,
          sidx_v, didx_v, bufs, tbl_sh, acc_sh, sems,
          ma_v, mb_v, mq_v, mt_v):
    c = lax.axis_index("c")
    s = lax.axis_index("s")
    wid = s * NC + c
    stripe = pl.ds(s * ZROWS, ZROWS)
    pltpu.sync_copy(zeros_hbm.at[stripe], acc_sh.at[stripe])
    _stage_indices(edge_hbm, sidx_v, didx_v, wid)
    pltpu.sync_copy(parts1_hbm.at[0, stripe], ma_v)
    pltpu.sync_copy(parts1_hbm.at[1, stripe], mb_v)
    pltpu.sync_copy(q1_hbm.at[stripe], mq_v)
    pltpu.sync_copy(zeros_hbm.at[stripe], mt_v)

    iota = lax.broadcasted_iota(jnp.int32, (L,), 0)

    def block_fn(blk, carry):
      ridx = jnp.minimum(blk * L + iota, ZROWS - 1)

      def col(ref, j):
        return plsc.load_gather(ref, [ridx, jnp.full((L,), j, jnp.int32)])

      deg = col(ma_v, D_HID) + col(mb_v, D_HID)
      rdeg = 1.0 / jnp.maximum(deg, 1.0)
      plsc.store_scatter(mt_v, [ridx, jnp.full((L,), D_HID, jnp.int32)],
                         rdeg)
      for j in range(D_HID):
        agg = (col(ma_v, j) + col(mb_v, j)) * rdeg
        h1 = jnp.maximum(agg + col(mq_v, j), 0.0)
        plsc.store_scatter(mt_v, [ridx, jnp.full((L,), j, jnp.int32)], h1)
      return carry

    lax.fori_loop(0, (ZROWS + L - 1) // L, block_fn, 0)
    pltpu.sync_copy(mt_v, tbl_sh.at[stripe])

    @pl.when(c == 0)
    def _():
      pltpu.sync_copy(mt_v, t2_hbm.at[stripe])

    plsc.subcore_barrier()
    _edge_loop(sidx_v, didx_v, bufs, sems, tbl_sh, acc_sh)
    plsc.subcore_barrier()
    pltpu.sync_copy(acc_sh.at[stripe], out_hbm.at[c, stripe])

  return sc2


def _l1_body(x_ref, wl_ref, wr_ref, b_ref, t_ref, q_ref):
  w = jnp.concatenate([wl_ref[...], wr_ref[...]], axis=1)
  h = jnp.dot(x_ref[...], w, preferred_element_type=jnp.float32,
              precision=lax.Precision.HIGHEST)
  ones = jnp.ones((ROWS_BLK, 1), jnp.float32)
  zeros7 = jnp.zeros((ROWS_BLK, TW - D_HID - 1), jnp.float32)
  t_ref[...] = jnp.concatenate([h[:, 0:D_HID], ones, zeros7], axis=1)
  zeros8 = jnp.zeros((ROWS_BLK, TW - D_HID), jnp.float32)
  q_ref[...] = jnp.concatenate(
      [h[:, D_HID:2 * D_HID] + b_ref[...], zeros8], axis=1)


def _out_body(parts_ref, t2_ref, w2l_ref, w2r_ref, b2_ref, o_ref):
  ssum = parts_ref[0] + parts_ref[1]
  rdeg = t2_ref[:, D_HID:D_HID + 1]
  h1 = t2_ref[:, 0:D_HID]
  agg2 = ssum[:, 0:D_HID] * rdeg
  logits = (jnp.dot(agg2, w2l_ref[...], preferred_element_type=jnp.float32,
                    precision=lax.Precision.HIGHEST)
            + jnp.dot(h1, w2r_ref[...], preferred_element_type=jnp.float32,
                      precision=lax.Precision.HIGHEST)
            + b2_ref[...])
  m = jnp.max(logits, axis=1, keepdims=True)
  z = logits - m
  lse = jnp.log(jnp.sum(jnp.exp(z), axis=1, keepdims=True))
  o_ref[...] = z - lse


_row_spec = lambda blk, w: pl.BlockSpec((blk, w), lambda i: (i, 0))
_full_spec = lambda r, w: pl.BlockSpec((r, w), lambda i: (0, 0))
_parts_spec = lambda blk: pl.BlockSpec((NC, blk, TW), lambda i: (0, i, 0))

_l1_call = pl.pallas_call(
    _l1_body,
    grid=(GRID,),
    in_specs=[_row_spec(ROWS_BLK, D_IN), _full_spec(D_IN, D_HID),
              _full_spec(D_IN, D_HID), _full_spec(1, D_HID)],
    out_specs=[_row_spec(ROWS_BLK, TW), _row_spec(ROWS_BLK, TW)],
    out_shape=[jax.ShapeDtypeStruct((ACC_ROWS, TW), jnp.float32),
               jax.ShapeDtypeStruct((ACC_ROWS, TW), jnp.float32)],
)

_out_call = pl.pallas_call(
    _out_body,
    grid=(OUT_GRID,),
    in_specs=[_parts_spec(OUT_BLK), _row_spec(OUT_BLK, TW),
              _full_spec(D_HID, N_CLASSES), _full_spec(D_HID, N_CLASSES),
              _full_spec(1, N_CLASSES)],
    out_specs=_row_spec(OUT_BLK, N_CLASSES),
    out_shape=jax.ShapeDtypeStruct((N, N_CLASSES), jnp.float32),
)


def kernel(x, edge_index, W1l, W1r, b1, W2l, W2r, b2):
  edges = edge_index.astype(jnp.int32)
  zeros_acc = jnp.zeros((ACC_ROWS, TW), jnp.float32)
  t1, q1 = _l1_call(x, W1l, W1r, b1.reshape(1, D_HID))
  parts1 = _sc_layer1()(t1, edges, zeros_acc)
  parts2, t2 = _sc_layer2()(parts1, q1, edges, zeros_acc)
  return _out_call(parts2, t2, W2l, W2r, b2.reshape(1, N_CLASSES))


# speedup vs baseline: 1.1507x; 1.0031x over previous
import functools

import jax
import jax.numpy as jnp
from jax import lax
from jax.experimental import pallas as pl
from jax.experimental.pallas import tpu as pltpu
from jax.experimental.pallas import tpu_sc as plsc

N = 10000
E = 320000
D_IN = 128
D_HID = 8
N_CLASSES = 40

NC = 2
NS = 16
NW = NC * NS
L = 16

EPW = E // NW
CHUNK = 512
CPW = 20
FULLW = (EPW // CHUNK) * CHUNK
TAILW = EPW - FULLW
SLOTS = CPW * CHUNK
TW = 16
ACC_ROWS = 10112
DUMMY = N
ZROWS = ACC_ROWS // NS

Q = 2
GRID = 2
ROWS_BLK = ACC_ROWS // GRID
OUT_GRID = 2
OUT_BLK = N // OUT_GRID

_MESH = dict(core_axis_name="c", subcore_axis_name="s", num_cores=NC,
             num_subcores=NS)
_SC_PARAMS = pltpu.CompilerParams(use_tc_tiling_on_sc=False,
                                  needs_layout_passes=False)


def _stage_indices(edge_hbm, sidx_v, didx_v, wid):
  base = wid * EPW
  pltpu.sync_copy(edge_hbm.at[0, pl.ds(base, FULLW)],
                  sidx_v.at[pl.ds(0, FULLW)])
  pltpu.sync_copy(edge_hbm.at[0, pl.ds(base + FULLW, TAILW)],
                  sidx_v.at[pl.ds(FULLW, TAILW)])
  pltpu.sync_copy(edge_hbm.at[1, pl.ds(base, FULLW)],
                  didx_v.at[pl.ds(0, FULLW)])
  pltpu.sync_copy(edge_hbm.at[1, pl.ds(base + FULLW, TAILW)],
                  didx_v.at[pl.ds(FULLW, TAILW)])
  for k in range(EPW, SLOTS, L):
    sidx_v[pl.ds(k, L)] = jnp.zeros((L,), jnp.int32)
    didx_v[pl.ds(k, L)] = jnp.full((L,), DUMMY, jnp.int32)


def _edge_loop(sidx_v, didx_v, bufs, sems, tbl_sh, acc_sh):
  grp = (tuple(bufs[:Q]), tuple(bufs[Q:]))
  gsem = (sems[0], sems[1])
  ssem = (sems[2], sems[3])

  def islice(iv, j):
    return iv.at[pl.ds(pl.multiple_of(j * CHUNK, CHUNK), CHUNK)]

  def fire_gathers(g, jbase):
    for i in range(Q):
      jn = jnp.minimum(jbase + i, CPW - 1)
      pltpu.async_copy(tbl_sh.at[islice(sidx_v, jn)], grp[g][i], gsem[g])

  def drain_gathers(g):
    for i in range(Q):
      pltpu.make_async_copy(tbl_sh.at[islice(sidx_v, 0)], grp[g][i],
                            gsem[g]).wait()

  def fire_scatters(g, jbase):
    for i in range(Q):
      pltpu.async_copy(grp[g][i], acc_sh.at[islice(didx_v, jbase + i)],
                       ssem[g], add=True)

  def drain_scatters(g):
    for i in range(Q):
      pltpu.make_async_copy(grp[g][i], acc_sh.at[islice(didx_v, 0)],
                            ssem[g]).wait()

  fire_gathers(0, 0)

  def round_fn(r, carry):
    j0 = r * 2 * Q

    @pl.when(r > 0)
    def _():
      drain_scatters(1)
    fire_gathers(1, j0 + Q)
    drain_gathers(0)
    fire_scatters(0, j0)

    drain_scatters(0)
    fire_gathers(0, j0 + 2 * Q)
    drain_gathers(1)
    fire_scatters(1, j0 + Q)
    return carry

  lax.fori_loop(0, CPW // (2 * Q), round_fn, 0)
  drain_scatters(1)
  drain_gathers(0)


_IDX_SCRATCH = [pltpu.VMEM((SLOTS,), jnp.int32),
                pltpu.VMEM((SLOTS,), jnp.int32)]
_BUF_SCRATCH = [pltpu.VMEM((CHUNK, TW), jnp.float32) for _ in range(2 * Q)]
_SEM_SCRATCH = [pltpu.SemaphoreType.DMA for _ in range(4)]


@functools.cache
def _sc_layer1():
  @functools.partial(
      pl.kernel,
      out_type=jax.ShapeDtypeStruct((NC, ACC_ROWS, TW), jnp.float32),
      mesh=plsc.VectorSubcoreMesh(**_MESH),
      scratch_types=[
          *_IDX_SCRATCH, _BUF_SCRATCH,
          pltpu.VMEM_SHARED((ACC_ROWS, TW), jnp.float32),
          pltpu.VMEM_SHARED((ACC_ROWS, TW), jnp.float32),
          _SEM_SCRATCH,
      ],
      compiler_params=_SC_PARAMS,
  )
  def sc1(table_hbm, edge_hbm, zeros_hbm, out_hbm,
          sidx_v, didx_v, bufs, tbl_sh, acc_sh, sems):
    c = lax.axis_index("c")
    s = lax.axis_index("s")
    wid = s * NC + c
    stripe = pl.ds(s * ZROWS, ZROWS)
    pltpu.sync_copy(table_hbm.at[stripe], tbl_sh.at[stripe])
    pltpu.sync_copy(zeros_hbm.at[stripe], acc_sh.at[stripe])
    _stage_indices(edge_hbm, sidx_v, didx_v, wid)
    plsc.subcore_barrier()
    _edge_loop(sidx_v, didx_v, bufs, sems, tbl_sh, acc_sh)
    plsc.subcore_barrier()
    pltpu.sync_copy(acc_sh.at[stripe], out_hbm.at[c, stripe])

  return sc1


@functools.cache
def _sc_layer2():
  @functools.partial(
      pl.kernel,
      out_type=(jax.ShapeDtypeStruct((NC, ACC_ROWS, TW), jnp.float32),
                jax.ShapeDtypeStruct((ACC_ROWS, TW), jnp.float32)),
      mesh=plsc.VectorSubcoreMesh(**_MESH),
      scratch_types=[
          *_IDX_SCRATCH, _BUF_SCRATCH,
          pltpu.VMEM_SHARED((ACC_ROWS, TW), jnp.float32),
          pltpu.VMEM_SHARED((ACC_ROWS, TW), jnp.float32),
          _SEM_SCRATCH,
          pltpu.VMEM((ZROWS, TW), jnp.float32),
          pltpu.VMEM((ZROWS, TW), jnp.float32),
          pltpu.VMEM((ZROWS, TW), jnp.float32),
          pltpu.VMEM((ZROWS, TW), jnp.float32),
      ],
      compiler_params=_SC_PARAMS,
  )
  def sc2(parts1_hbm, q1_hbm, edge_hbm, zeros_hbm, out_hbm, t2_hbm,
          sidx_v, didx_v, bufs, tbl_sh, acc_sh, sems,
          ma_v, mb_v, mq_v, mt_v):
    c = lax.axis_index("c")
    s = lax.axis_index("s")
    wid = s * NC + c
    stripe = pl.ds(s * ZROWS, ZROWS)
    pltpu.sync_copy(zeros_hbm.at[stripe], acc_sh.at[stripe])
    _stage_indices(edge_hbm, sidx_v, didx_v, wid)
    pltpu.sync_copy(parts1_hbm.at[0, stripe], ma_v)
    pltpu.sync_copy(parts1_hbm.at[1, stripe], mb_v)
    pltpu.sync_copy(q1_hbm.at[stripe], mq_v)
    pltpu.sync_copy(zeros_hbm.at[stripe], mt_v)

    iota = lax.broadcasted_iota(jnp.int32, (L,), 0)

    def block_fn(blk, carry):
      ridx = jnp.minimum(blk * L + iota, ZROWS - 1)

      def col(ref, j):
        return plsc.load_gather(ref, [ridx, jnp.full((L,), j, jnp.int32)])

      deg = col(ma_v, D_HID) + col(mb_v, D_HID)
      rdeg = 1.0 / jnp.maximum(deg, 1.0)
      plsc.store_scatter(mt_v, [ridx, jnp.full((L,), D_HID, jnp.int32)],
                         rdeg)
      for j in range(D_HID):
        agg = (col(ma_v, j) + col(mb_v, j)) * rdeg
        h1 = jnp.maximum(agg + col(mq_v, j), 0.0)
        plsc.store_scatter(mt_v, [ridx, jnp.full((L,), j, jnp.int32)], h1)
      return carry

    lax.fori_loop(0, (ZROWS + L - 1) // L, block_fn, 0)
    pltpu.sync_copy(mt_v, tbl_sh.at[stripe])

    @pl.when(c == 0)
    def _():
      pltpu.sync_copy(mt_v, t2_hbm.at[stripe])

    plsc.subcore_barrier()
    _edge_loop(sidx_v, didx_v, bufs, sems, tbl_sh, acc_sh)
    plsc.subcore_barrier()
    pltpu.sync_copy(acc_sh.at[stripe], out_hbm.at[c, stripe])

  return sc2


def _l1_body(x_ref, wl_ref, wr_ref, b_ref, t_ref, q_ref):
  w = jnp.concatenate([wl_ref[...], wr_ref[...]], axis=1)
  h = jnp.dot(x_ref[...], w, preferred_element_type=jnp.float32,
              precision=lax.Precision.HIGHEST)
  ones = jnp.ones((ROWS_BLK, 1), jnp.float32)
  zeros7 = jnp.zeros((ROWS_BLK, TW - D_HID - 1), jnp.float32)
  t_ref[...] = jnp.concatenate([h[:, 0:D_HID], ones, zeros7], axis=1)
  zeros8 = jnp.zeros((ROWS_BLK, TW - D_HID), jnp.float32)
  q_ref[...] = jnp.concatenate(
      [h[:, D_HID:2 * D_HID] + b_ref[...], zeros8], axis=1)


def _out_body(parts_ref, t2_ref, w2l_ref, w2r_ref, b2_ref, o_ref):
  ssum = parts_ref[0] + parts_ref[1]
  rdeg = t2_ref[:, D_HID:D_HID + 1]
  h1 = t2_ref[:, 0:D_HID]
  agg2 = ssum[:, 0:D_HID] * rdeg
  logits = (jnp.dot(agg2, w2l_ref[...], preferred_element_type=jnp.float32,
                    precision=lax.Precision.HIGHEST)
            + jnp.dot(h1, w2r_ref[...], preferred_element_type=jnp.float32,
                      precision=lax.Precision.HIGHEST)
            + b2_ref[...])
  m = jnp.max(logits, axis=1, keepdims=True)
  z = logits - m
  lse = jnp.log(jnp.sum(jnp.exp(z), axis=1, keepdims=True))
  o_ref[...] = z - lse


_row_spec = lambda blk, w: pl.BlockSpec((blk, w), lambda i: (i, 0))
_full_spec = lambda r, w: pl.BlockSpec((r, w), lambda i: (0, 0))

_l1_call = pl.pallas_call(
    _l1_body,
    grid=(GRID,),
    in_specs=[_row_spec(ROWS_BLK, D_IN), _full_spec(D_IN, D_HID),
              _full_spec(D_IN, D_HID), _full_spec(1, D_HID)],
    out_specs=[_row_spec(ROWS_BLK, TW), _row_spec(ROWS_BLK, TW)],
    out_shape=[jax.ShapeDtypeStruct((ACC_ROWS, TW), jnp.float32),
               jax.ShapeDtypeStruct((ACC_ROWS, TW), jnp.float32)],
)

_parts_spec = lambda blk: pl.BlockSpec((NC, blk, TW), lambda i: (0, i, 0))

_out_call = pl.pallas_call(
    _out_body,
    grid=(OUT_GRID,),
    in_specs=[_parts_spec(OUT_BLK), _row_spec(OUT_BLK, TW),
              _full_spec(D_HID, N_CLASSES), _full_spec(D_HID, N_CLASSES),
              _full_spec(1, N_CLASSES)],
    out_specs=_row_spec(OUT_BLK, N_CLASSES),
    out_shape=jax.ShapeDtypeStruct((N, N_CLASSES), jnp.float32),
)


def kernel(x, edge_index, W1l, W1r, b1, W2l, W2r, b2):
  edges = edge_index.astype(jnp.int32)
  zeros_acc = jnp.zeros((ACC_ROWS, TW), jnp.float32)
  t1, q1 = _l1_call(x, W1l, W1r, b1.reshape(1, D_HID))
  parts1 = _sc_layer1()(t1, edges, zeros_acc)
  parts2, t2 = _sc_layer2()(parts1, q1, edges, zeros_acc)
  return _out_call(parts2, t2, W2l, W2r, b2.reshape(1, N_CLASSES))
